# Initial kernel scaffold; baseline (speedup 1.0000x reference)
import jax
import jax.numpy as jnp
from jax.experimental import pallas as pl


def kernel(W, TmT, Y, taus, Btab, Ctab):
    raise NotImplementedError("write your pallas kernel here")


# speedup vs baseline: 2.7080x; 2.7080x over previous
import functools

import jax
import jax.numpy as jnp
from jax import lax
from jax.experimental import pallas as pl
from jax.experimental.pallas import tpu as pltpu
from jax.experimental.pallas import tpu_sc as plsc

_ALPHA = 0.8
_GAMMA = 5.0
_T = 1.5
_PI_CAP = 2.0
_RHO = 0.3
_SIGMA = 0.2
_SIGMA_Y = 0.3

_L = 16
_NC = 2
_NS = 16
_NW = _NC * _NS


def _tec_body(K, per_w, tmt_hbm, y_hbm, bt_hbm, ct_hbm, out_hbm,
              tmt_v, y_v, out_v, bt_v, ct_v):
    wid = lax.axis_index("s") * _NC + lax.axis_index("c")
    base = wid * per_w
    pltpu.sync_copy(bt_hbm, bt_v)
    pltpu.sync_copy(ct_hbm, ct_v)
    pltpu.sync_copy(tmt_hbm.at[pl.ds(base, per_w)], tmt_v)
    pltpu.sync_copy(y_hbm.at[pl.ds(base, per_w)], y_v)

    scale = jnp.float32((K - 1) / _T)
    a_s = jnp.float32(_ALPHA / _SIGMA)
    rss = jnp.float32(_RHO * _SIGMA_Y / _SIGMA)
    inv_g = jnp.float32(1.0 / _GAMMA)

    def body(j, carry):
        sl = pl.ds(j * jnp.int32(_L), _L)
        t = tmt_v[sl]
        t = jnp.minimum(jnp.maximum(t, jnp.float32(0.0)), jnp.float32(_T))
        s = t * scale
        i0 = jnp.minimum(s.astype(jnp.int32), K - 2)
        f = s - i0.astype(jnp.float32)
        i1 = i0 + 1
        b0 = plsc.load_gather(bt_v, [i0])
        b1 = plsc.load_gather(bt_v, [i1])
        c0 = plsc.load_gather(ct_v, [i0])
        c1 = plsc.load_gather(ct_v, [i1])
        b = b0 + f * (b1 - b0)
        c = c0 + f * (c1 - c0)
        y = y_v[sl]
        pi = inv_g * (a_s * y + rss * (b + c * y))
        out_v[sl] = jnp.minimum(jnp.maximum(pi, jnp.float32(-_PI_CAP)),
                                jnp.float32(_PI_CAP))
        return carry

    lax.fori_loop(jnp.int32(0), jnp.int32(per_w // _L), body, jnp.int32(0))
    pltpu.sync_copy(out_v, out_hbm.at[pl.ds(base, per_w)])


def kernel(W, TmT, Y, taus, Btab, Ctab):
    del W
    N = TmT.shape[0]
    K = taus.shape[0]
    per_w = N // _NW
    mesh = plsc.VectorSubcoreMesh(core_axis_name="c", subcore_axis_name="s")
    run = pl.kernel(
        functools.partial(_tec_body, K, per_w),
        mesh=mesh,
        compiler_params=pltpu.CompilerParams(needs_layout_passes=False),
        out_type=jax.ShapeDtypeStruct((N,), jnp.float32),
        scratch_types=[
            pltpu.VMEM((per_w,), jnp.float32),
            pltpu.VMEM((per_w,), jnp.float32),
            pltpu.VMEM((per_w,), jnp.float32),
            pltpu.VMEM((K,), jnp.float32),
            pltpu.VMEM((K,), jnp.float32),
        ],
    )
    out = run(TmT.astype(jnp.float32), Y.reshape(N).astype(jnp.float32),
              Btab.reshape(K).astype(jnp.float32),
              Ctab.reshape(K).astype(jnp.float32))
    return out.reshape(N, 1)


# speedup vs baseline: 4.4944x; 1.6597x over previous
import functools

import jax
import jax.numpy as jnp
from jax import lax
from jax.experimental import pallas as pl
from jax.experimental.pallas import tpu as pltpu
from jax.experimental.pallas import tpu_sc as plsc

_ALPHA = 0.8
_GAMMA = 5.0
_T = 1.5
_PI_CAP = 2.0
_RHO = 0.3
_SIGMA = 0.2
_SIGMA_Y = 0.3

_L = 16
_NC = 2
_NS = 16
_NW = _NC * _NS


def _tec_body(K, per_w, tmt_hbm, y_hbm, bt_hbm, ct_hbm, out_hbm,
              tmt_v, y_v, out_v, bt_v, ct_v):
    wid = lax.axis_index("s") * _NC + lax.axis_index("c")
    base = wid * per_w
    pltpu.sync_copy(bt_hbm, bt_v)
    pltpu.sync_copy(ct_hbm, ct_v)
    pltpu.sync_copy(tmt_hbm.at[pl.ds(base, per_w)], tmt_v)
    pltpu.sync_copy(y_hbm.at[pl.ds(base, per_w)], y_v)

    scale = jnp.float32((K - 1) / _T)
    k1 = jnp.float32(_ALPHA / _SIGMA / _GAMMA)
    k2 = jnp.float32(_RHO * _SIGMA_Y / _SIGMA / _GAMMA)
    bt_v[:] = bt_v[:] * k2
    ct_v[:] = ct_v[:] * k2

    @plsc.parallel_loop(jnp.int32(0), jnp.int32(per_w), step=jnp.int32(_L),
                        unroll=8)
    def body(off):
        sl = pl.ds(off, _L)
        t = tmt_v[sl]
        t = jnp.minimum(jnp.maximum(t, jnp.float32(0.0)), jnp.float32(_T))
        s = t * scale
        i0 = jnp.minimum(s.astype(jnp.int32), K - 2)
        fr = s - i0.astype(jnp.float32)
        i1 = i0 + 1
        b0 = plsc.load_gather(bt_v, [i0])
        b1 = plsc.load_gather(bt_v, [i1])
        c0 = plsc.load_gather(ct_v, [i0])
        c1 = plsc.load_gather(ct_v, [i1])
        b = b0 + fr * (b1 - b0)
        c = c0 + fr * (c1 - c0)
        y = y_v[sl]
        pi = k1 * y + (b + c * y)
        out_v[sl] = jnp.minimum(jnp.maximum(pi, jnp.float32(-_PI_CAP)),
                                jnp.float32(_PI_CAP))
    pltpu.sync_copy(out_v, out_hbm.at[pl.ds(base, per_w)])


def kernel(W, TmT, Y, taus, Btab, Ctab):
    del W
    N = TmT.shape[0]
    K = taus.shape[0]
    per_w = N // _NW
    mesh = plsc.VectorSubcoreMesh(core_axis_name="c", subcore_axis_name="s")
    run = pl.kernel(
        functools.partial(_tec_body, K, per_w),
        mesh=mesh,
        compiler_params=pltpu.CompilerParams(needs_layout_passes=False),
        out_type=jax.ShapeDtypeStruct((N,), jnp.float32),
        scratch_types=[
            pltpu.VMEM((per_w,), jnp.float32),
            pltpu.VMEM((per_w,), jnp.float32),
            pltpu.VMEM((per_w,), jnp.float32),
            pltpu.VMEM((K,), jnp.float32),
            pltpu.VMEM((K,), jnp.float32),
        ],
    )
    out = run(TmT.astype(jnp.float32), Y.reshape(N).astype(jnp.float32),
              Btab.reshape(K).astype(jnp.float32),
              Ctab.reshape(K).astype(jnp.float32))
    return out.reshape(N, 1)
